# baseline (speedup 1.0000x reference)
import functools

import jax
import jax.numpy as jnp
from jax import lax
from jax.experimental import pallas as pl
from jax.experimental.pallas import tpu as pltpu
from jax.experimental.pallas import tpu_sc as plsc


@functools.lru_cache(maxsize=None)
def _build(B, D, N, NC, NS):
    NW = NC * NS
    b_per_w = B // NW
    SIZES = (16, 32, 64, 128, 128, 96, 48)
    assert sum(SIZES) == b_per_w and max(SIZES) <= 128
    OFFS = tuple(sum(SIZES[:i]) for i in range(len(SIZES)))
    NCH = len(SIZES)
    CMAX = max(SIZES)
    NSLOT = 3

    mesh = plsc.VectorSubcoreMesh(core_axis_name="c", subcore_axis_name="s")

    @functools.partial(
        pl.kernel,
        mesh=mesh,
        out_type=(
            jax.ShapeDtypeStruct((B, D), jnp.float32),
            jax.ShapeDtypeStruct((B, D), jnp.float32),
            jax.ShapeDtypeStruct((B,), jnp.float32),
        ),
        scratch_types=(
            [pltpu.VMEM((b_per_w,), jnp.int32),
             pltpu.VMEM((b_per_w,), jnp.float32)]
            + [pltpu.VMEM((CMAX, D), jnp.float32) for _ in range(2 * NSLOT)]
            + [pltpu.VMEM((CMAX,), jnp.float32) for _ in range(NSLOT)]
            + [pltpu.SemaphoreType.DMA for _ in range(2 * NSLOT)]
        ),
    )
    def k(nids_hbm, ts_hbm, left_hbm, right_hbm, pts_hbm,
          hl_out, hr_out, pts_out,
          idx_v, ts_v, *rest):
        rowbufs = rest[:2 * NSLOT]
        ptsbufs = rest[2 * NSLOT:3 * NSLOT]
        gsems = rest[3 * NSLOT:4 * NSLOT]
        wsems = rest[4 * NSLOT:5 * NSLOT]
        bufs = tuple((rowbufs[2 * s], rowbufs[2 * s + 1], ptsbufs[s])
                     for s in range(NSLOT))
        wid = lax.axis_index("s") * NC + lax.axis_index("c")
        base = wid * b_per_w
        head = OFFS[2]
        pltpu.sync_copy(nids_hbm.at[pl.ds(base, head)],
                        idx_v.at[pl.ds(0, head)])
        icp = pltpu.async_copy(nids_hbm.at[pl.ds(base + head, b_per_w - head)],
                               idx_v.at[pl.ds(head, b_per_w - head)],
                               wsems[1])
        tscp = pltpu.async_copy(ts_hbm.at[pl.ds(base, b_per_w)], ts_v,
                                wsems[0])

        def fire_gather(j, slot):
            c, off = SIZES[j], OFFS[j]
            ij = idx_v.at[pl.ds(off, c)]
            l, r, p = bufs[slot]
            return (pltpu.async_copy(left_hbm.at[ij], l.at[pl.ds(0, c)],
                                     gsems[slot]),
                    pltpu.async_copy(right_hbm.at[ij], r.at[pl.ds(0, c)],
                                     gsems[slot]),
                    pltpu.async_copy(pts_hbm.at[ij], p.at[pl.ds(0, c)],
                                     gsems[slot]))

        AHEAD = NSLOT - 1
        pending_g = [None] * NSLOT
        pending_w = [None] * NSLOT
        for j0 in range(min(AHEAD, NCH)):
            pending_g[j0 % NSLOT] = fire_gather(j0, j0 % NSLOT)
        icp.wait()
        tscp.wait()
        for j in range(NCH):
            slot = j % NSLOT
            f = j + AHEAD
            if f < NCH:
                fslot = f % NSLOT
                if pending_w[fslot] is not None:
                    for cp in pending_w[fslot]:
                        cp.wait()
                    pending_w[fslot] = None
                pending_g[fslot] = fire_gather(f, fslot)
            for cp in pending_g[slot]:
                cp.wait()
            c, off = SIZES[j], OFFS[j]
            l, r, p = bufs[slot]
            for i in range(c // 16):
                sl = pl.ds(i * 16, 16)
                p[sl] = jnp.minimum(p[sl], ts_v[pl.ds(off + i * 16, 16)])
            o = pl.ds(base + off, c)
            pending_w[slot] = (
                pltpu.async_copy(l.at[pl.ds(0, c)], hl_out.at[o],
                                 wsems[slot]),
                pltpu.async_copy(r.at[pl.ds(0, c)], hr_out.at[o],
                                 wsems[slot]),
                pltpu.async_copy(p.at[pl.ds(0, c)], pts_out.at[o],
                                 wsems[slot]))
        for pw in pending_w:
            if pw is not None:
                for cp in pw:
                    cp.wait()

    return k


def kernel(nids, ts, left_weight, right_weight, prev_ts_table):
    B, = nids.shape
    N, D = left_weight.shape
    info = plsc.get_sparse_core_info()
    k = _build(B, D, N, info.num_cores, info.num_subcores)
    h_left, h_right, prev_ts = k(nids.astype(jnp.int32), ts, left_weight,
                                 right_weight, prev_ts_table)
    return (h_left, h_right, prev_ts)


# speedup vs baseline: 1.0269x; 1.0269x over previous
import functools

import jax
import jax.numpy as jnp
from jax import lax
from jax.experimental import pallas as pl
from jax.experimental.pallas import tpu as pltpu
from jax.experimental.pallas import tpu_sc as plsc


@functools.lru_cache(maxsize=None)
def _build(B, D, N, NC, NS):
    NW = NC * NS
    b_per_w = B // NW
    C = 128
    NCH = b_per_w // C
    NSLOT = 3

    mesh = plsc.VectorSubcoreMesh(core_axis_name="c", subcore_axis_name="s")

    @functools.partial(
        pl.kernel,
        mesh=mesh,
        out_type=(
            jax.ShapeDtypeStruct((B, D), jnp.float32),
            jax.ShapeDtypeStruct((B, D), jnp.float32),
            jax.ShapeDtypeStruct((B,), jnp.float32),
        ),
        scratch_types=(
            [pltpu.VMEM((NCH, C), jnp.int32),
             pltpu.VMEM((b_per_w,), jnp.float32),
             pltpu.VMEM((b_per_w,), jnp.float32)]
            + [pltpu.VMEM((C, D), jnp.float32) for _ in range(2 * NSLOT)]
            + [pltpu.SemaphoreType.DMA for _ in range(2 * NSLOT)]
            + [pltpu.SemaphoreType.DMA]
        ),
    )
    def k(nids_hbm, ts_hbm, left_hbm, right_hbm, pts_hbm,
          hl_out, hr_out, pts_out,
          idx_v, ts_v, pts_v, *rest):
        rowbufs = rest[:2 * NSLOT]
        gsems = rest[2 * NSLOT:3 * NSLOT]
        wsems = rest[3 * NSLOT:4 * NSLOT]
        psem = rest[4 * NSLOT]
        bufs = tuple((rowbufs[2 * s], rowbufs[2 * s + 1])
                     for s in range(NSLOT))
        wid = lax.axis_index("s") * NC + lax.axis_index("c")
        base = wid * b_per_w
        pltpu.sync_copy(nids_hbm.at[wid], idx_v)

        def fire_gather(j, slot):
            ij = idx_v.at[j]
            l, r = bufs[slot]
            return (pltpu.async_copy(left_hbm.at[ij], l, gsems[slot]),
                    pltpu.async_copy(right_hbm.at[ij], r, gsems[slot]))

        AHEAD = NSLOT - 1
        pending_g = [None] * NSLOT
        pending_w = [None] * NSLOT
        for j0 in range(min(AHEAD, NCH)):
            pending_g[j0 % NSLOT] = fire_gather(j0, j0 % NSLOT)

        pcps = [pltpu.async_copy(pts_hbm.at[idx_v.at[j]],
                                 pts_v.at[pl.ds(j * C, C)], psem)
                for j in range(NCH)]
        tscp = pltpu.async_copy(ts_hbm.at[pl.ds(base, b_per_w)], ts_v, psem)
        for cp in pcps:
            cp.wait()
        tscp.wait()
        for i in range(b_per_w // 16):
            sl = pl.ds(i * 16, 16)
            pts_v[sl] = jnp.minimum(pts_v[sl], ts_v[sl])
        pwcp = pltpu.async_copy(pts_v, pts_out.at[pl.ds(base, b_per_w)], psem)

        for j in range(NCH):
            slot = j % NSLOT
            f = j + AHEAD
            if f < NCH:
                fslot = f % NSLOT
                if pending_w[fslot] is not None:
                    for cp in pending_w[fslot]:
                        cp.wait()
                    pending_w[fslot] = None
                pending_g[fslot] = fire_gather(f, fslot)
            for cp in pending_g[slot]:
                cp.wait()
            l, r = bufs[slot]
            o = pl.ds(base + j * C, C)
            pending_w[slot] = (
                pltpu.async_copy(l, hl_out.at[o], wsems[slot]),
                pltpu.async_copy(r, hr_out.at[o], wsems[slot]))
        pwcp.wait()
        for pw in pending_w:
            if pw is not None:
                for cp in pw:
                    cp.wait()

    return k, NW, NCH, C


def kernel(nids, ts, left_weight, right_weight, prev_ts_table):
    B, = nids.shape
    N, D = left_weight.shape
    info = plsc.get_sparse_core_info()
    k, NW, NCH, C = _build(B, D, N, info.num_cores, info.num_subcores)
    nids3 = nids.astype(jnp.int32).reshape(NW, NCH, C)
    h_left, h_right, prev_ts = k(nids3, ts, left_weight, right_weight,
                                 prev_ts_table)
    return (h_left, h_right, prev_ts)
